# baseline (speedup 1.0000x reference)
import jax
import jax.numpy as jnp
from jax import lax
from jax.experimental import pallas as pl
from jax.experimental.pallas import tpu as pltpu
from jax.experimental.pallas import tpu_sc as plsc

NUM_USERS = 30000
NUM_ITEMS = 20000
N_NODES = NUM_USERS + NUM_ITEMS
EMB = 64
NNZ = 400000
NUM_LAYERS = 3
BATCH = 4096

NC = 2
NS = 16
CHUNK = 128
NCHUNK = 200
PAD = NS * NCHUNK * CHUNK - NNZ
GROUP = 8
NGROUP = NCHUNK // GROUP
GARBAGE_ROW = 30000
ACC_ROWS = 30008
ZROWS = 200
NZCH0 = NUM_USERS // ZROWS
NZCH1 = NUM_ITEMS // ZROWS

_MESH = plsc.VectorSubcoreMesh(core_axis_name="c", subcore_axis_name="s")
_CP = pltpu.CompilerParams(use_tc_tiling_on_sc=False)


def _rowwise(c, s, fn):
    nch = jnp.where(c == 0, NZCH0, NZCH1)

    @pl.loop(0, (NZCH0 + NS - 1) // NS)
    def _(j):
        chunk = j * NS + s

        @pl.when(chunk < nch)
        def _():
            fn(chunk * ZROWS)


HEMB = EMB // 2
N_PAD = N_NODES + 8
SUBCH = 50
SGROUP = 5
SNGRP = SUBCH // SGROUP


def _spmm_body(f_hbm, cols_hbm, rows_hbm, zeros_hbm, out_hbm,
               acc, colb, rowb, gbuf, gsem0, gsem1, ssem0, ssem1, isem):
    c = lax.axis_index("c")
    s = lax.axis_index("s")
    gsem = (gsem0, gsem1)
    ssem = (ssem0, ssem1)

    GR = SGROUP * CHUNK

    def fire_gath(p, g, st):
        pltpu.async_copy(f_hbm.at[p].at[colb.at[pl.ds(g * GR, GR)]],
                         gbuf.at[st], gsem[st])

    def fire_scat(g, st):
        for j in range(SGROUP):
            pltpu.async_copy(gbuf.at[st, pl.ds(j * CHUNK, CHUNK)],
                             acc.at[rowb.at[g * SGROUP + j]],
                             ssem[st], add=True)

    def drain(sem, st):
        for j in range(SGROUP):
            pltpu.make_async_copy(f_hbm.at[0, pl.ds(0, CHUNK)],
                                  gbuf.at[st, pl.ds(j * CHUNK, CHUNK)], sem).wait()

    for p in range(2):
        _rowwise(c, s, lambda r: pltpu.sync_copy(zeros_hbm, acc.at[pl.ds(r, ZROWS)]))
        if p == 0:

            @pl.when((c == 1) & (s == NS - 1))
            def _():
                pltpu.sync_copy(zeros_hbm.at[pl.ds(0, 8)],
                                out_hbm.at[0, pl.ds(N_NODES, 8)])
                pltpu.sync_copy(zeros_hbm.at[pl.ds(0, 8)],
                                out_hbm.at[1, pl.ds(N_NODES, 8)])

        plsc.subcore_barrier()

        for sb in range(NCHUNK // SUBCH):
            base = sb * SUBCH
            i1 = pltpu.async_copy(
                cols_hbm.at[c, s, pl.ds(base * CHUNK, SUBCH * CHUNK)], colb, isem)
            i2 = pltpu.async_copy(rows_hbm.at[c, s, pl.ds(base, SUBCH)], rowb, isem)
            i1.wait()
            i2.wait()
            fire_gath(p, 0, 0)
            drain(gsem[0], 0)
            fire_gath(p, 1, 1)
            fire_scat(0, 0)

            @pl.loop(0, SNGRP // 2 - 1)
            def _(i):
                for off in (2, 3):
                    g = 2 * i + off
                    cur = off % 2
                    nxt = 1 - cur
                    drain(ssem[cur], cur)
                    drain(gsem[nxt], nxt)
                    fire_gath(p, g, cur)
                    fire_scat(g - 1, nxt)

            drain(gsem[1], 1)
            fire_scat(SNGRP - 1, 1)
            drain(ssem[0], 0)
            drain(ssem[1], 1)

        plsc.subcore_barrier()

        off_r = jnp.where(c == 0, 0, NUM_USERS)
        _rowwise(c, s, lambda r: pltpu.sync_copy(
            acc.at[pl.ds(r, ZROWS)], out_hbm.at[p, pl.ds(off_r + r, ZROWS)]))

        if p == 0:
            plsc.subcore_barrier()


_spmm = pl.kernel(
    _spmm_body,
    out_type=jax.ShapeDtypeStruct((2, N_PAD, HEMB), jnp.float32),
    mesh=_MESH,
    compiler_params=_CP,
    scratch_types=[
        pltpu.VMEM_SHARED((ACC_ROWS, HEMB), jnp.float32),
        pltpu.VMEM((SUBCH * CHUNK,), jnp.int32),
        pltpu.VMEM((SUBCH, CHUNK), jnp.int32),
        pltpu.VMEM((2, SGROUP * CHUNK, HEMB), jnp.float32),
        pltpu.SemaphoreType.DMA,
        pltpu.SemaphoreType.DMA,
        pltpu.SemaphoreType.DMA,
        pltpu.SemaphoreType.DMA,
        pltpu.SemaphoreType.DMA,
    ],
)


def _deg_body(rows_hbm, ones_hbm, zeros_hbm, out_hbm,
              accd, onesb, zrow, rowb, ssem):
    c = lax.axis_index("c")
    s = lax.axis_index("s")

    pltpu.sync_copy(ones_hbm, onesb)
    pltpu.sync_copy(zeros_hbm, zrow)
    _rowwise(c, s, lambda r: pltpu.sync_copy(zrow, accd.at[pl.ds(r, ZROWS)]))

    plsc.subcore_barrier()

    @pl.loop(0, NGROUP)
    def _(g):
        pltpu.sync_copy(rows_hbm.at[c, s, pl.ds(g * GROUP, GROUP)], rowb)
        scatters = [
            pltpu.async_copy(onesb, accd.at[rowb.at[j]], ssem, add=True)
            for j in range(GROUP)
        ]
        for cp in scatters:
            cp.wait()

    plsc.subcore_barrier()

    off = jnp.where(c == 0, 0, NUM_USERS)
    _rowwise(c, s, lambda r: pltpu.sync_copy(
        accd.at[pl.ds(r, ZROWS)], out_hbm.at[pl.ds(off + r, ZROWS)]))


_deg = pl.kernel(
    _deg_body,
    out_type=jax.ShapeDtypeStruct((N_NODES, 16), jnp.float32),
    mesh=_MESH,
    compiler_params=_CP,
    scratch_types=[
        pltpu.VMEM_SHARED((ACC_ROWS, 16), jnp.float32),
        pltpu.VMEM((CHUNK, 16), jnp.float32),
        pltpu.VMEM((ZROWS, 16), jnp.float32),
        pltpu.VMEM((GROUP, CHUNK), jnp.int32),
        pltpu.SemaphoreType.DMA,
    ],
)

_B_CHUNKS = 3 * BATCH // (NC * NS * CHUNK)


def _bgather_body(tab_hbm, idx_hbm, out_hbm, idxb, gbuf, gsem):
    c = lax.axis_index("c")
    s = lax.axis_index("s")
    wid = c * NS + s
    pltpu.sync_copy(idx_hbm.at[c, s], idxb)
    gathers = [
        pltpu.async_copy(tab_hbm.at[idxb.at[j]], gbuf.at[j], gsem)
        for j in range(_B_CHUNKS)
    ]
    for cp in gathers:
        cp.wait()
    pltpu.sync_copy(gbuf, out_hbm.at[pl.ds(wid * _B_CHUNKS, _B_CHUNKS)])


_bgather = pl.kernel(
    _bgather_body,
    out_type=jax.ShapeDtypeStruct((NC * NS * _B_CHUNKS, CHUNK, EMB), jnp.float32),
    mesh=_MESH,
    compiler_params=_CP,
    scratch_types=[
        pltpu.VMEM((_B_CHUNKS, CHUNK), jnp.int32),
        pltpu.VMEM((_B_CHUNKS, CHUNK, EMB), jnp.float32),
        pltpu.SemaphoreType.DMA,
    ],
)


def kernel(users, pos_items, neg_items, user_emb, item_emb,
           adj_rows, adj_cols, adj_vals):
    del adj_vals

    ego = jnp.concatenate([user_emb, item_emb], axis=0)

    pad_rows = jnp.full((PAD,), GARBAGE_ROW, jnp.int32)
    pad_cols = jnp.full((PAD,), N_NODES, jnp.int32)
    rows3 = jnp.stack([
        jnp.concatenate([adj_rows[:NNZ], pad_rows]).reshape(NS, NCHUNK, CHUNK),
        jnp.concatenate([adj_rows[NNZ:] - NUM_USERS, pad_rows]).reshape(NS, NCHUNK, CHUNK),
    ])
    cols3 = jnp.stack([
        jnp.concatenate([adj_cols[:NNZ], pad_cols]).reshape(NS, NCHUNK * CHUNK),
        jnp.concatenate([adj_cols[NNZ:], pad_cols]).reshape(NS, NCHUNK * CHUNK),
    ])

    ones16 = jnp.ones((CHUNK, 16), jnp.float32)
    zeros16 = jnp.zeros((ZROWS, 16), jnp.float32)
    zeros32 = jnp.zeros((ZROWS, HEMB), jnp.float32)

    deg = _deg(rows3, ones16, zeros16)[:, 0]
    dis = jnp.where(deg > 0, lax.rsqrt(jnp.maximum(deg, 1.0)), 0.0)
    zpad8 = jnp.zeros((8,), jnp.float32)
    disext = jnp.concatenate([dis, zpad8])[None, :, None]
    dis2ext = disext * disext

    zpad32 = jnp.zeros((8, HEMB), jnp.float32)
    ego_sp = jnp.stack([
        jnp.concatenate([ego[:, :HEMB], zpad32], axis=0),
        jnp.concatenate([ego[:, HEMB:], zpad32], axis=0),
    ])
    acc = ego_sp
    f = ego_sp * disext
    for _ in range(NUM_LAYERS):
        seg = _spmm(f, cols3, rows3, zeros32)
        acc = acc + seg * disext
        f = seg * dis2ext
    final = jnp.concatenate([acc[0, :N_NODES], acc[1, :N_NODES]], axis=1) * 0.25

    idx = jnp.concatenate([
        users.astype(jnp.int32),
        pos_items.astype(jnp.int32) + NUM_USERS,
        neg_items.astype(jnp.int32) + NUM_USERS,
    ]).reshape(NC, NS, _B_CHUNKS, CHUNK)
    g = _bgather(final, idx).reshape(3 * BATCH, EMB)

    return (g[:BATCH], g[BATCH:2 * BATCH], g[2 * BATCH:], final[NUM_USERS:])


# speedup vs baseline: 1.0312x; 1.0312x over previous
import jax
import jax.numpy as jnp
from jax import lax
from jax.experimental import pallas as pl
from jax.experimental.pallas import tpu as pltpu
from jax.experimental.pallas import tpu_sc as plsc

NUM_USERS = 30000
NUM_ITEMS = 20000
N_NODES = NUM_USERS + NUM_ITEMS
EMB = 64
NNZ = 400000
NUM_LAYERS = 3
BATCH = 4096

NC = 2
NS = 16
CHUNK = 128
NCHUNK = 200
PAD = NS * NCHUNK * CHUNK - NNZ
GROUP = 8
NGROUP = NCHUNK // GROUP
GARBAGE_ROW = 30000
ACC_ROWS = 30008
ZROWS = 400
NZCH0 = NUM_USERS // ZROWS
NZCH1 = NUM_ITEMS // ZROWS

_MESH = plsc.VectorSubcoreMesh(core_axis_name="c", subcore_axis_name="s")
_CP = pltpu.CompilerParams(use_tc_tiling_on_sc=False)


def _rowwise(c, s, fn):
    nch = jnp.where(c == 0, NZCH0, NZCH1)

    @pl.loop(0, (NZCH0 + NS - 1) // NS)
    def _(j):
        chunk = j * NS + s

        @pl.when(chunk < nch)
        def _():
            fn(chunk * ZROWS)


HEMB = EMB // 2
N_PAD = N_NODES + 8
SUBCH = 40
SGROUP = 5
SNGRP = SUBCH // SGROUP


def _spmm_body(f_hbm, cols_hbm, rows_hbm, zeros_hbm, dis2x_hbm,
               raw_hbm, fnext_hbm,
               acc, colb, rowb, gbuf, tbuf, dbuf,
               gsem0, gsem1, ssem0, ssem1, isem):
    c = lax.axis_index("c")
    s = lax.axis_index("s")
    gsem = (gsem0, gsem1)
    ssem = (ssem0, ssem1)

    GR = SGROUP * CHUNK

    def fire_gath(p, g, st):
        pltpu.async_copy(f_hbm.at[p].at[colb.at[pl.ds(g * GR, GR)]],
                         gbuf.at[st], gsem[st])

    def fire_scat(g, st):
        for j in range(SGROUP):
            pltpu.async_copy(gbuf.at[st, pl.ds(j * CHUNK, CHUNK)],
                             acc.at[rowb.at[g * SGROUP + j]],
                             ssem[st], add=True)

    def drain(sem, st):
        for j in range(SGROUP):
            pltpu.make_async_copy(f_hbm.at[0, pl.ds(0, CHUNK)],
                                  gbuf.at[st, pl.ds(j * CHUNK, CHUNK)], sem).wait()

    for p in range(2):
        _rowwise(c, s, lambda r: pltpu.sync_copy(zeros_hbm, acc.at[pl.ds(r, ZROWS)]))
        if p == 0:

            @pl.when((c == 1) & (s == NS - 1))
            def _():
                pltpu.sync_copy(zeros_hbm.at[pl.ds(0, 8)],
                                fnext_hbm.at[0, pl.ds(N_NODES, 8)])
                pltpu.sync_copy(zeros_hbm.at[pl.ds(0, 8)],
                                fnext_hbm.at[1, pl.ds(N_NODES, 8)])

        plsc.subcore_barrier()

        for sb in range(NCHUNK // SUBCH):
            base = sb * SUBCH
            i1 = pltpu.async_copy(
                cols_hbm.at[c, s, pl.ds(base * CHUNK, SUBCH * CHUNK)], colb, isem)
            i2 = pltpu.async_copy(rows_hbm.at[c, s, pl.ds(base, SUBCH)], rowb, isem)
            i1.wait()
            i2.wait()
            fire_gath(p, 0, 0)
            drain(gsem[0], 0)
            fire_gath(p, 1, 1)
            fire_scat(0, 0)

            @pl.loop(0, SNGRP // 2 - 1)
            def _(i):
                for off in (2, 3):
                    g = 2 * i + off
                    cur = off % 2
                    nxt = 1 - cur
                    drain(ssem[cur], cur)
                    drain(gsem[nxt], nxt)
                    fire_gath(p, g, cur)
                    fire_scat(g - 1, nxt)

            drain(gsem[1], 1)
            fire_scat(SNGRP - 1, 1)
            drain(ssem[0], 0)
            drain(ssem[1], 1)

        plsc.subcore_barrier()

        off_r = jnp.where(c == 0, 0, NUM_USERS)

        def copy_out(r):
            ro = off_r + r
            pltpu.sync_copy(acc.at[pl.ds(r, ZROWS)], tbuf)
            pltpu.sync_copy(tbuf, raw_hbm.at[p, pl.ds(ro, ZROWS)])
            pltpu.sync_copy(dis2x_hbm.at[pl.ds(ro, ZROWS)], dbuf)

            @pl.loop(0, ZROWS)
            def _(i):
                dv = dbuf[i, pl.ds(0, 16)]
                tbuf[i, pl.ds(0, 16)] = tbuf[i, pl.ds(0, 16)] * dv
                tbuf[i, pl.ds(16, 16)] = tbuf[i, pl.ds(16, 16)] * dv

            pltpu.sync_copy(tbuf, fnext_hbm.at[p, pl.ds(ro, ZROWS)])

        _rowwise(c, s, copy_out)

        if p == 0:
            plsc.subcore_barrier()


_spmm = pl.kernel(
    _spmm_body,
    out_type=(jax.ShapeDtypeStruct((2, N_PAD, HEMB), jnp.float32),
              jax.ShapeDtypeStruct((2, N_PAD, HEMB), jnp.float32)),
    mesh=_MESH,
    compiler_params=_CP,
    scratch_types=[
        pltpu.VMEM_SHARED((ACC_ROWS, HEMB), jnp.float32),
        pltpu.VMEM((SUBCH * CHUNK,), jnp.int32),
        pltpu.VMEM((SUBCH, CHUNK), jnp.int32),
        pltpu.VMEM((2, SGROUP * CHUNK, HEMB), jnp.float32),
        pltpu.VMEM((ZROWS, HEMB), jnp.float32),
        pltpu.VMEM((ZROWS, 16), jnp.float32),
        pltpu.SemaphoreType.DMA,
        pltpu.SemaphoreType.DMA,
        pltpu.SemaphoreType.DMA,
        pltpu.SemaphoreType.DMA,
        pltpu.SemaphoreType.DMA,
    ],
)


def _deg_body(rows_hbm, ones_hbm, zeros_hbm, out_hbm,
              accd, onesb, zrow, rowb, ssem):
    c = lax.axis_index("c")
    s = lax.axis_index("s")

    pltpu.sync_copy(ones_hbm, onesb)
    pltpu.sync_copy(zeros_hbm, zrow)
    _rowwise(c, s, lambda r: pltpu.sync_copy(zrow, accd.at[pl.ds(r, ZROWS)]))

    plsc.subcore_barrier()

    @pl.loop(0, NGROUP)
    def _(g):
        pltpu.sync_copy(rows_hbm.at[c, s, pl.ds(g * GROUP, GROUP)], rowb)
        scatters = [
            pltpu.async_copy(onesb, accd.at[rowb.at[j]], ssem, add=True)
            for j in range(GROUP)
        ]
        for cp in scatters:
            cp.wait()

    plsc.subcore_barrier()

    off = jnp.where(c == 0, 0, NUM_USERS)
    _rowwise(c, s, lambda r: pltpu.sync_copy(
        accd.at[pl.ds(r, ZROWS)], out_hbm.at[pl.ds(off + r, ZROWS)]))


_deg = pl.kernel(
    _deg_body,
    out_type=jax.ShapeDtypeStruct((N_NODES, 16), jnp.float32),
    mesh=_MESH,
    compiler_params=_CP,
    scratch_types=[
        pltpu.VMEM_SHARED((ACC_ROWS, 16), jnp.float32),
        pltpu.VMEM((CHUNK, 16), jnp.float32),
        pltpu.VMEM((ZROWS, 16), jnp.float32),
        pltpu.VMEM((GROUP, CHUNK), jnp.int32),
        pltpu.SemaphoreType.DMA,
    ],
)

_B_CHUNKS = 3 * BATCH // (NC * NS * CHUNK)


def _bgather_body(tab_hbm, idx_hbm, out_hbm, idxb, gbuf, gsem):
    c = lax.axis_index("c")
    s = lax.axis_index("s")
    wid = c * NS + s
    pltpu.sync_copy(idx_hbm.at[c, s], idxb)
    gathers = [
        pltpu.async_copy(tab_hbm.at[idxb.at[j]], gbuf.at[j], gsem)
        for j in range(_B_CHUNKS)
    ]
    for cp in gathers:
        cp.wait()
    pltpu.sync_copy(gbuf, out_hbm.at[pl.ds(wid * _B_CHUNKS, _B_CHUNKS)])


_bgather = pl.kernel(
    _bgather_body,
    out_type=jax.ShapeDtypeStruct((NC * NS * _B_CHUNKS, CHUNK, EMB), jnp.float32),
    mesh=_MESH,
    compiler_params=_CP,
    scratch_types=[
        pltpu.VMEM((_B_CHUNKS, CHUNK), jnp.int32),
        pltpu.VMEM((_B_CHUNKS, CHUNK, EMB), jnp.float32),
        pltpu.SemaphoreType.DMA,
    ],
)


def kernel(users, pos_items, neg_items, user_emb, item_emb,
           adj_rows, adj_cols, adj_vals):
    del adj_vals

    ego = jnp.concatenate([user_emb, item_emb], axis=0)

    pad_rows = jnp.full((PAD,), GARBAGE_ROW, jnp.int32)
    pad_cols = jnp.full((PAD,), N_NODES, jnp.int32)
    rows3 = jnp.stack([
        jnp.concatenate([adj_rows[:NNZ], pad_rows]).reshape(NS, NCHUNK, CHUNK),
        jnp.concatenate([adj_rows[NNZ:] - NUM_USERS, pad_rows]).reshape(NS, NCHUNK, CHUNK),
    ])
    cols3 = jnp.stack([
        jnp.concatenate([adj_cols[:NNZ], pad_cols]).reshape(NS, NCHUNK * CHUNK),
        jnp.concatenate([adj_cols[NNZ:], pad_cols]).reshape(NS, NCHUNK * CHUNK),
    ])

    ones16 = jnp.ones((CHUNK, 16), jnp.float32)
    zeros16 = jnp.zeros((ZROWS, 16), jnp.float32)
    zeros32 = jnp.zeros((ZROWS, HEMB), jnp.float32)

    deg = _deg(rows3, ones16, zeros16)[:, 0]
    dis = jnp.where(deg > 0, lax.rsqrt(jnp.maximum(deg, 1.0)), 0.0)
    zpad8 = jnp.zeros((8,), jnp.float32)
    disext = jnp.concatenate([dis, zpad8])[None, :, None]
    dis2x = jnp.broadcast_to((dis * dis)[:, None], (N_NODES, 16))
    dis2x = jnp.concatenate([dis2x, jnp.zeros((8, 16), jnp.float32)], axis=0)

    zpad32 = jnp.zeros((8, HEMB), jnp.float32)
    ego_sp = jnp.stack([
        jnp.concatenate([ego[:, :HEMB], zpad32], axis=0),
        jnp.concatenate([ego[:, HEMB:], zpad32], axis=0),
    ])
    f = ego_sp * disext
    ssum = None
    for _ in range(NUM_LAYERS):
        raw, fnext = _spmm(f, cols3, rows3, zeros32, dis2x)
        ssum = raw if ssum is None else ssum + raw
        f = fnext
    acc = ego_sp + ssum * disext
    final = jnp.concatenate([acc[0, :N_NODES], acc[1, :N_NODES]], axis=1) * 0.25

    idx = jnp.concatenate([
        users.astype(jnp.int32),
        pos_items.astype(jnp.int32) + NUM_USERS,
        neg_items.astype(jnp.int32) + NUM_USERS,
    ]).reshape(NC, NS, _B_CHUNKS, CHUNK)
    g = _bgather(final, idx).reshape(3 * BATCH, EMB)

    return (g[:BATCH], g[BATCH:2 * BATCH], g[2 * BATCH:], final[NUM_USERS:])


# speedup vs baseline: 1.5630x; 1.5157x over previous
import jax
import jax.numpy as jnp
from jax import lax
from jax.experimental import pallas as pl
from jax.experimental.pallas import tpu as pltpu
from jax.experimental.pallas import tpu_sc as plsc

NUM_USERS = 30000
NUM_ITEMS = 20000
N_NODES = NUM_USERS + NUM_ITEMS
EMB = 64
NNZ = 400000
NUM_LAYERS = 3
BATCH = 4096

NC = 2
NS = 16
CHUNK = 128
NCHUNK = 200
PAD = NS * NCHUNK * CHUNK - NNZ
GROUP = 8
NGROUP = NCHUNK // GROUP
ZROWS = 80
NZCH0 = NUM_USERS // ZROWS
NZCH1 = NUM_ITEMS // ZROWS

_MESH = plsc.VectorSubcoreMesh(core_axis_name="c", subcore_axis_name="s")
_CP = pltpu.CompilerParams(use_tc_tiling_on_sc=False)


def _rowwise(c, s, fn):
    nch = jnp.where(c == 0, NZCH0, NZCH1)

    @pl.loop(0, (NZCH0 + NS - 1) // NS)
    def _(j):
        chunk = j * NS + s

        @pl.when(chunk < nch)
        def _():
            fn(chunk * ZROWS)


HEMB = EMB // 2
N_PAD = N_NODES + 8
SUBCH = 20
SGROUP = 2
SNGRP = SUBCH // SGROUP

S_TOTAL = 50016
SRC0_BASE = 30008
SRC1_BASE = 20008
GARB0 = 30000
GARB1 = 20000
SRC_PAD = 50008
NSRC0 = NUM_ITEMS // ZROWS
NSRC1 = NUM_USERS // ZROWS


def _spmm_body(f_hbm, cols_hbm, rows_hbm, zeros_hbm, dis2x_hbm,
               raw_hbm, fnext_hbm,
               shr, colb, rowb, gbuf, tbuf, dbuf, zrow,
               gsem0, gsem1, ssem0, ssem1, isem):
    c = lax.axis_index("c")
    s = lax.axis_index("s")
    gsem = (gsem0, gsem1)
    ssem = (ssem0, ssem1)

    GR = SGROUP * CHUNK

    pltpu.sync_copy(zeros_hbm, zrow)

    nsrc = jnp.where(c == 0, NSRC0, NSRC1)
    hbase = jnp.where(c == 0, NUM_USERS, 0)
    sbase = jnp.where(c == 0, SRC0_BASE, SRC1_BASE)

    def fire_gath(g, st):
        pltpu.async_copy(shr.at[colb.at[pl.ds(g * GR, GR)]],
                         gbuf.at[st], gsem[st])

    def fire_scat(g, st):
        for j in range(SGROUP):
            pltpu.async_copy(gbuf.at[st, pl.ds(j * CHUNK, CHUNK)],
                             shr.at[rowb.at[g * SGROUP + j]],
                             ssem[st], add=True)

    def drain(sem, st):
        for j in range(SGROUP):
            pltpu.make_async_copy(shr.at[pl.ds(0, CHUNK)],
                                  gbuf.at[st, pl.ds(j * CHUNK, CHUNK)], sem).wait()

    for p in range(2):
        _rowwise(c, s, lambda r: pltpu.sync_copy(zrow, shr.at[pl.ds(r, ZROWS)]))

        @pl.loop(0, (NSRC1 + NS - 1) // NS)
        def _(j):
            ch = j * NS + s

            @pl.when(ch < nsrc)
            def _():
                r = ch * ZROWS
                pltpu.sync_copy(f_hbm.at[p, pl.ds(hbase + r, ZROWS)],
                                shr.at[pl.ds(sbase + r, ZROWS)])

        plsc.subcore_barrier()

        for sb in range(NCHUNK // SUBCH):
            base = sb * SUBCH
            i1 = pltpu.async_copy(
                cols_hbm.at[c, s, pl.ds(base * CHUNK, SUBCH * CHUNK)], colb, isem)
            i2 = pltpu.async_copy(rows_hbm.at[c, s, pl.ds(base, SUBCH)], rowb, isem)
            i1.wait()
            i2.wait()
            fire_gath(0, 0)
            drain(gsem[0], 0)
            fire_gath(1, 1)
            fire_scat(0, 0)

            @pl.loop(0, SNGRP // 2 - 1)
            def _(i):
                for off in (2, 3):
                    g = 2 * i + off
                    cur = off % 2
                    nxt = 1 - cur
                    drain(ssem[cur], cur)
                    drain(gsem[nxt], nxt)
                    fire_gath(g, cur)
                    fire_scat(g - 1, nxt)

            drain(gsem[1], 1)
            fire_scat(SNGRP - 1, 1)
            drain(ssem[0], 0)
            drain(ssem[1], 1)

        plsc.subcore_barrier()

        off_r = jnp.where(c == 0, 0, NUM_USERS)

        def copy_out(r):
            ro = off_r + r
            pltpu.sync_copy(shr.at[pl.ds(r, ZROWS)], tbuf)
            pltpu.sync_copy(tbuf, raw_hbm.at[p, pl.ds(ro, ZROWS)])
            pltpu.sync_copy(dis2x_hbm.at[pl.ds(ro, ZROWS)], dbuf)

            @pl.loop(0, ZROWS)
            def _(i):
                dv = dbuf[i, pl.ds(0, 16)]
                tbuf[i, pl.ds(0, 16)] = tbuf[i, pl.ds(0, 16)] * dv
                tbuf[i, pl.ds(16, 16)] = tbuf[i, pl.ds(16, 16)] * dv

            pltpu.sync_copy(tbuf, fnext_hbm.at[p, pl.ds(ro, ZROWS)])

        _rowwise(c, s, copy_out)

        if p == 0:
            plsc.subcore_barrier()


_spmm = pl.kernel(
    _spmm_body,
    out_type=(jax.ShapeDtypeStruct((2, N_PAD, HEMB), jnp.float32),
              jax.ShapeDtypeStruct((2, N_PAD, HEMB), jnp.float32)),
    mesh=_MESH,
    compiler_params=_CP,
    scratch_types=[
        pltpu.VMEM_SHARED((S_TOTAL, HEMB), jnp.float32),
        pltpu.VMEM((SUBCH * CHUNK,), jnp.int32),
        pltpu.VMEM((SUBCH, CHUNK), jnp.int32),
        pltpu.VMEM((2, SGROUP * CHUNK, HEMB), jnp.float32),
        pltpu.VMEM((ZROWS, HEMB), jnp.float32),
        pltpu.VMEM((ZROWS, 16), jnp.float32),
        pltpu.VMEM((ZROWS, HEMB), jnp.float32),
        pltpu.SemaphoreType.DMA,
        pltpu.SemaphoreType.DMA,
        pltpu.SemaphoreType.DMA,
        pltpu.SemaphoreType.DMA,
        pltpu.SemaphoreType.DMA,
    ],
)

ACC_ROWS = 30008


def _deg_body(rows_hbm, ones_hbm, zeros_hbm, out_hbm,
              accd, onesb, zrow, rowb, ssem):
    c = lax.axis_index("c")
    s = lax.axis_index("s")

    pltpu.sync_copy(ones_hbm, onesb)
    pltpu.sync_copy(zeros_hbm, zrow)
    _rowwise(c, s, lambda r: pltpu.sync_copy(zrow, accd.at[pl.ds(r, ZROWS)]))

    plsc.subcore_barrier()

    @pl.loop(0, NGROUP)
    def _(g):
        pltpu.sync_copy(rows_hbm.at[c, s, pl.ds(g * GROUP, GROUP)], rowb)
        scatters = [
            pltpu.async_copy(onesb, accd.at[rowb.at[j]], ssem, add=True)
            for j in range(GROUP)
        ]
        for cp in scatters:
            cp.wait()

    plsc.subcore_barrier()

    off = jnp.where(c == 0, 0, NUM_USERS)
    _rowwise(c, s, lambda r: pltpu.sync_copy(
        accd.at[pl.ds(r, ZROWS)], out_hbm.at[pl.ds(off + r, ZROWS)]))


_deg = pl.kernel(
    _deg_body,
    out_type=jax.ShapeDtypeStruct((N_NODES, 16), jnp.float32),
    mesh=_MESH,
    compiler_params=_CP,
    scratch_types=[
        pltpu.VMEM_SHARED((ACC_ROWS, 16), jnp.float32),
        pltpu.VMEM((CHUNK, 16), jnp.float32),
        pltpu.VMEM((ZROWS, 16), jnp.float32),
        pltpu.VMEM((GROUP, CHUNK), jnp.int32),
        pltpu.SemaphoreType.DMA,
    ],
)

_B_CHUNKS = 3 * BATCH // (NC * NS * CHUNK)


def _bgather_body(tab_hbm, idx_hbm, out_hbm, idxb, gbuf, gsem):
    c = lax.axis_index("c")
    s = lax.axis_index("s")
    wid = c * NS + s
    pltpu.sync_copy(idx_hbm.at[c, s], idxb)
    gathers = [
        pltpu.async_copy(tab_hbm.at[idxb.at[j]], gbuf.at[j], gsem)
        for j in range(_B_CHUNKS)
    ]
    for cp in gathers:
        cp.wait()
    pltpu.sync_copy(gbuf, out_hbm.at[pl.ds(wid * _B_CHUNKS, _B_CHUNKS)])


_bgather = pl.kernel(
    _bgather_body,
    out_type=jax.ShapeDtypeStruct((NC * NS * _B_CHUNKS, CHUNK, EMB), jnp.float32),
    mesh=_MESH,
    compiler_params=_CP,
    scratch_types=[
        pltpu.VMEM((_B_CHUNKS, CHUNK), jnp.int32),
        pltpu.VMEM((_B_CHUNKS, CHUNK, EMB), jnp.float32),
        pltpu.SemaphoreType.DMA,
    ],
)


def kernel(users, pos_items, neg_items, user_emb, item_emb,
           adj_rows, adj_cols, adj_vals):
    del adj_vals

    ego = jnp.concatenate([user_emb, item_emb], axis=0)

    rows3 = jnp.stack([
        jnp.concatenate([adj_rows[:NNZ],
                         jnp.full((PAD,), GARB0, jnp.int32)]).reshape(NS, NCHUNK, CHUNK),
        jnp.concatenate([adj_rows[NNZ:] - NUM_USERS,
                         jnp.full((PAD,), GARB1, jnp.int32)]).reshape(NS, NCHUNK, CHUNK),
    ])
    pad_cols = jnp.full((PAD,), SRC_PAD, jnp.int32)
    cols3 = jnp.stack([
        jnp.concatenate([adj_cols[:NNZ] + (SRC0_BASE - NUM_USERS),
                         pad_cols]).reshape(NS, NCHUNK * CHUNK),
        jnp.concatenate([adj_cols[NNZ:] + SRC1_BASE,
                         pad_cols]).reshape(NS, NCHUNK * CHUNK),
    ])

    ones16 = jnp.ones((CHUNK, 16), jnp.float32)
    zeros16 = jnp.zeros((ZROWS, 16), jnp.float32)
    zeros32 = jnp.zeros((ZROWS, HEMB), jnp.float32)

    deg = _deg(rows3, ones16, zeros16)[:, 0]
    dis = jnp.where(deg > 0, lax.rsqrt(jnp.maximum(deg, 1.0)), 0.0)
    zpad8 = jnp.zeros((8,), jnp.float32)
    disext = jnp.concatenate([dis, zpad8])[None, :, None]
    dis2x = jnp.broadcast_to((dis * dis)[:, None], (N_NODES, 16))
    dis2x = jnp.concatenate([dis2x, jnp.zeros((8, 16), jnp.float32)], axis=0)

    zpad32 = jnp.zeros((8, HEMB), jnp.float32)
    ego_sp = jnp.stack([
        jnp.concatenate([ego[:, :HEMB], zpad32], axis=0),
        jnp.concatenate([ego[:, HEMB:], zpad32], axis=0),
    ])
    f = ego_sp * disext
    ssum = None
    for _ in range(NUM_LAYERS):
        raw, fnext = _spmm(f, cols3, rows3, zeros32, dis2x)
        ssum = raw if ssum is None else ssum + raw
        f = fnext
    acc = ego_sp + ssum * disext
    final = jnp.concatenate([acc[0, :N_NODES], acc[1, :N_NODES]], axis=1) * 0.25

    idx = jnp.concatenate([
        users.astype(jnp.int32),
        pos_items.astype(jnp.int32) + NUM_USERS,
        neg_items.astype(jnp.int32) + NUM_USERS,
    ]).reshape(NC, NS, _B_CHUNKS, CHUNK)
    g = _bgather(final, idx).reshape(3 * BATCH, EMB)

    return (g[:BATCH], g[BATCH:2 * BATCH], g[2 * BATCH:], final[NUM_USERS:])
